# baseline (speedup 1.0000x reference)
import functools

import jax
import jax.numpy as jnp
from jax import lax
from jax.experimental import pallas as pl
from jax.experimental.pallas import tpu as pltpu
from jax.experimental.pallas import tpu_sc as plsc

NC = 2
NS = 16
L = 16
NW = NC * NS

B = 16384
D = 16
BPW = B // NW
NCHUNK = 4
CHUNK = BPW // NCHUNK


def _svdpp_body(x_hbm, ue_hbm, ie_hbm, ub_hbm, ib_hbm, out_hbm,
                x_v, uidx_v, iidx_v, urows_v, irows_v, ub_v, ib_v, out_v,
                sem):
    wid = lax.axis_index("s") * NC + lax.axis_index("c")
    base = wid * BPW

    pltpu.sync_copy(x_hbm.at[pl.ds(base * 2, BPW * 2)], x_v)

    iota = lax.iota(jnp.int32, L)

    for j in range(NCHUNK):
        for i in range(CHUNK // L):
            flat = (iota + (j * CHUNK + i * L)) * 2
            uidx_v[j, pl.ds(i * L, L)] = plsc.load_gather(x_v, [flat])
            iidx_v[j, pl.ds(i * L, L)] = plsc.load_gather(x_v, [flat + 1])

    copies = []
    for j in range(NCHUNK):
        copies.append(pltpu.async_copy(
            ue_hbm.at[uidx_v.at[j]], urows_v.at[pl.ds(j * CHUNK, CHUNK)], sem))
        copies.append(pltpu.async_copy(
            ie_hbm.at[iidx_v.at[j]], irows_v.at[pl.ds(j * CHUNK, CHUNK)], sem))
        copies.append(pltpu.async_copy(
            ub_hbm.at[uidx_v.at[j]], ub_v.at[pl.ds(j * CHUNK, CHUNK)], sem))
        copies.append(pltpu.async_copy(
            ib_hbm.at[iidx_v.at[j]], ib_v.at[pl.ds(j * CHUNK, CHUNK)], sem))
    for c in copies:
        c.wait()

    for t in range(BPW // L):
        rows = iota + t * L
        acc = jnp.zeros((L,), jnp.float32)
        s = jnp.zeros((L,), jnp.float32)
        for c in range(D):
            cc = jnp.full((L,), c, jnp.int32)
            u = plsc.load_gather(urows_v, [rows, cc])
            v = plsc.load_gather(irows_v, [rows, cc])
            acc = acc + u * v
            s = s + u
        z = acc + ub_v[pl.ds(t * L, L)] + ib_v[pl.ds(t * L, L)] + s * (1.0 / D)
        out_v[pl.ds(t * L, L)] = 1.0 / (1.0 + jnp.exp(-z))

    pltpu.sync_copy(out_v, out_hbm.at[pl.ds(base, BPW)])


@jax.jit
def kernel(x, user_emb, item_emb, user_bias, item_bias):
    xf = x.reshape(-1)
    ub = user_bias.reshape(-1)
    ib = item_bias.reshape(-1)
    mesh = plsc.VectorSubcoreMesh(core_axis_name="c", subcore_axis_name="s",
                                  num_cores=NC, num_subcores=NS)
    run = pl.kernel(
        _svdpp_body,
        out_type=jax.ShapeDtypeStruct((B,), jnp.float32),
        mesh=mesh,
        compiler_params=pltpu.CompilerParams(needs_layout_passes=False,
                                             use_tc_tiling_on_sc=False),
        scratch_types=[
            pltpu.VMEM((BPW * 2,), jnp.int32),
            pltpu.VMEM((NCHUNK, CHUNK), jnp.int32),
            pltpu.VMEM((NCHUNK, CHUNK), jnp.int32),
            pltpu.VMEM((BPW, D), jnp.float32),
            pltpu.VMEM((BPW, D), jnp.float32),
            pltpu.VMEM((BPW,), jnp.float32),
            pltpu.VMEM((BPW,), jnp.float32),
            pltpu.VMEM((BPW,), jnp.float32),
            pltpu.SemaphoreType.DMA,
        ],
    )
    return run(xf, user_emb, item_emb, ub, ib)
